# Initial kernel scaffold; baseline (speedup 1.0000x reference)
import jax
import jax.numpy as jnp
from jax.experimental import pallas as pl


def kernel(T, H, W, temporal_embed, height_embed, width_embed):
    raise NotImplementedError("write your pallas kernel here")


# speedup vs baseline: 2.6922x; 2.6922x over previous
import jax
import jax.numpy as jnp
from jax.experimental import pallas as pl
from jax.experimental.pallas import tpu as pltpu

T_ST, H_ST, W_ST = 16, 64, 64
HIDDEN = 768
D3 = HIDDEN // 3
BH = 16


def _body(offs_ref, t_ref, h_ref, w_ref, out_ref):
    del offs_ref
    shape = (1, BH, W_ST, D3)
    t_vec = t_ref[0]
    h_rows = h_ref[:, :]
    w_rows = w_ref[:, :]
    out_ref[:, :, :, 0:D3] = jnp.broadcast_to(t_vec[:, None, None, :], shape)
    out_ref[:, :, :, D3:2 * D3] = jnp.broadcast_to(h_rows[None, :, None, :], shape)
    out_ref[:, :, :, 2 * D3:HIDDEN] = jnp.broadcast_to(w_rows[None, None, :, :], shape)


def kernel(T, H, W, temporal_embed, height_embed, width_embed):
    offs = jnp.stack([
        jnp.asarray(T, jnp.int32) - T_ST,
        jnp.asarray(H, jnp.int32) - H_ST,
        jnp.asarray(W, jnp.int32) - W_ST,
    ])
    grid = (T_ST, H_ST // BH)
    out4 = pl.pallas_call(
        _body,
        grid_spec=pltpu.PrefetchScalarGridSpec(
            num_scalar_prefetch=1,
            grid=grid,
            in_specs=[
                pl.BlockSpec((1, 1, D3), lambda t, hb, offs: (offs[0] + t, 0, 0)),
                pl.BlockSpec((BH, D3), lambda t, hb, offs: ((offs[1] + hb * BH) // BH, 0)),
                pl.BlockSpec((W_ST, D3), lambda t, hb, offs: (offs[2] // W_ST, 0)),
            ],
            out_specs=pl.BlockSpec(
                (1, BH, W_ST, HIDDEN), lambda t, hb, offs: (t, hb, 0, 0)
            ),
        ),
        out_shape=jax.ShapeDtypeStruct((T_ST, H_ST, W_ST, HIDDEN), jnp.float32),
    )(offs, temporal_embed.reshape(-1, 1, D3), height_embed, width_embed)
    return out4.reshape(T_ST * H_ST * W_ST, HIDDEN)


# speedup vs baseline: 2.9345x; 1.0900x over previous
import jax
import jax.numpy as jnp
from jax.experimental import pallas as pl
from jax.experimental.pallas import tpu as pltpu

T_ST, H_ST, W_ST = 16, 64, 64
HIDDEN = 768
D3 = HIDDEN // 3
BH = 16
NB = H_ST // BH
NBUF = 4
GRID = T_ST * NB


def _body(offs_ref, t_ref, h_ref, w_ref, out_ref, scratch, sem):
    del offs_ref
    i = pl.program_id(0)
    t = i // NB
    hb = i % NB
    buf = jax.lax.rem(i, NBUF)

    dst = out_ref.at[t, pl.ds(hb * BH, BH), :, :]

    @pl.when(i >= NBUF)
    def _wait_prev():
        pltpu.make_async_copy(scratch.at[buf], dst, sem.at[buf]).wait()

    shape = (BH, W_ST, D3)
    t_vec = t_ref[0]
    h_rows = h_ref[:, :]
    w_rows = w_ref[:, :]
    scratch[buf, :, :, 0:D3] = jnp.broadcast_to(t_vec[None, :, :], shape)
    scratch[buf, :, :, D3:2 * D3] = jnp.broadcast_to(h_rows[:, None, :], shape)
    scratch[buf, :, :, 2 * D3:HIDDEN] = jnp.broadcast_to(w_rows[None, :, :], shape)

    pltpu.make_async_copy(scratch.at[buf], dst, sem.at[buf]).start()

    @pl.when(i == GRID - 1)
    def _drain():
        for b in range(NBUF):
            pltpu.make_async_copy(scratch.at[b], dst, sem.at[b]).wait()


def kernel(T, H, W, temporal_embed, height_embed, width_embed):
    offs = jnp.stack([
        jnp.asarray(T, jnp.int32) - T_ST,
        jnp.asarray(H, jnp.int32) - H_ST,
        jnp.asarray(W, jnp.int32) - W_ST,
    ])
    out4 = pl.pallas_call(
        _body,
        grid_spec=pltpu.PrefetchScalarGridSpec(
            num_scalar_prefetch=1,
            grid=(GRID,),
            in_specs=[
                pl.BlockSpec((1, 1, D3), lambda i, offs: (offs[0] + i // NB, 0, 0)),
                pl.BlockSpec((BH, D3), lambda i, offs: ((offs[1] + (i % NB) * BH) // BH, 0)),
                pl.BlockSpec((W_ST, D3), lambda i, offs: (offs[2] // W_ST, 0)),
            ],
            out_specs=pl.BlockSpec(memory_space=pl.ANY),
            scratch_shapes=[
                pltpu.VMEM((NBUF, BH, W_ST, HIDDEN), jnp.float32),
                pltpu.SemaphoreType.DMA((NBUF,)),
            ],
        ),
        out_shape=jax.ShapeDtypeStruct((T_ST, H_ST, W_ST, HIDDEN), jnp.float32),
    )(offs, temporal_embed.reshape(-1, 1, D3), height_embed, width_embed)
    return out4.reshape(T_ST * H_ST * W_ST, HIDDEN)
